# Initial kernel scaffold; baseline (speedup 1.0000x reference)
import jax
import jax.numpy as jnp
from jax.experimental import pallas as pl


def kernel(lufeat, llabel, mbank, start, W1, b1, W2, b2, W3, b3, W4, b4, W5, b5, g1, be1, g2, be2):
    raise NotImplementedError("write your pallas kernel here")


# speedup vs baseline: 121.8145x; 121.8145x over previous
import functools

import jax
import jax.numpy as jnp
from jax import lax
from jax.experimental import pallas as pl
from jax.experimental.pallas import tpu as pltpu

_LN09 = -0.10536051565782628
_HI = lax.Precision.HIGHEST


def _tc_kernel(lufeat_ref, w1t_ref, b1_ref, w2t_ref, b2_ref,
               lblc_ref, lblr_ref, startc_ref, mbank_ref,
               w3ta_ref, w3tb_ref, b3_ref, g1_ref, be1_ref,
               w4t_ref, b4_ref, g2_ref, be2_ref, w5t_ref, b5_ref,
               lsc_ref, usc_ref, mbu_ref):
    f32 = jnp.float32
    h1 = jnp.dot(lufeat_ref[...], w1t_ref[...], preferred_element_type=f32) + b1_ref[...]
    h2 = jnp.dot(h1, w2t_ref[...], preferred_element_type=f32) + b2_ref[...]
    nrm = jnp.sqrt(jnp.sum(h2 * h2, axis=1, keepdims=True))
    h = h2 / jnp.maximum(nrm, 1e-12)
    lfeat = h[:512]
    ufeat = h[512:]

    lblc = lblc_ref[...]
    lblr = lblr_ref[...]
    startc = startc_ref[...]
    match = (lblc == lblr)
    jj = lax.broadcasted_iota(jnp.int32, (512, 512), 0)
    ii = lax.broadcasted_iota(jnp.int32, (512, 512), 1)
    pc = jnp.sum(jnp.where(match & (jj <= ii), 1.0, 0.0), axis=0, keepdims=True)
    cnt = jnp.sum(jnp.where(match, 1.0, 0.0), axis=0, keepdims=True)
    r = cnt - pc
    onehot_t = (lax.broadcasted_iota(jnp.int32, (512, 512), 0) == lblr)
    st_i = jnp.sum(jnp.where(onehot_t, startc, 0.0), axis=0, keepdims=True)
    first = (pc == 1.0) & (st_i == 0.0)
    coeff = jnp.exp(r * _LN09) * jnp.where(first, 1.0, 0.1)
    cnt_c = jnp.sum(jnp.where(onehot_t, 1.0, 0.0), axis=1, keepdims=True)
    base = jnp.where((startc == 0.0) & (cnt_c > 0.0), 0.0, jnp.exp(cnt_c * _LN09))
    scat = jnp.dot(jnp.where(onehot_t, coeff, 0.0), lfeat, preferred_element_type=f32, precision=_HI)
    mbu = base * mbank_ref[...] + scat
    mbu_ref[...] = mbu

    mn2 = jnp.sum(mbu * mbu, axis=1, keepdims=True)
    mext = jnp.concatenate([mbu * -2.0, mn2], axis=1)
    lext = jnp.concatenate([lfeat, jnp.ones((512, 1), f32)], axis=1)
    gl = lax.dot_general(lext, mext, (((1,), (1,)), ((), ())),
                         preferred_element_type=f32, precision=_HI)
    lm = jnp.sqrt(jnp.maximum(gl + 1.0, 0.0))
    lsc_ref[...] = jnp.min(lm, axis=1, keepdims=True)

    uext = jnp.concatenate([ufeat, jnp.ones((512, 1), f32)], axis=1)
    gu = lax.dot_general(uext, mext, (((1,), (1,)), ((), ())),
                         preferred_element_type=f32, precision=_HI)
    um = jnp.sqrt(jnp.maximum(gu + 1.0, 0.0))

    z = (jnp.dot(ufeat, w3ta_ref[...], preferred_element_type=f32)
         + jnp.dot(um, w3tb_ref[...], preferred_element_type=f32) + b3_ref[...])
    m1 = jnp.mean(z, axis=0, keepdims=True)
    v1 = jnp.mean((z - m1) * (z - m1), axis=0, keepdims=True)
    u1 = jnp.maximum(g1_ref[...] * (z - m1) / jnp.sqrt(v1 + 1e-5) + be1_ref[...], 0.0)
    z2 = jnp.dot(u1, w4t_ref[...], preferred_element_type=f32) + b4_ref[...]
    m2 = jnp.mean(z2, axis=0, keepdims=True)
    v2 = jnp.mean((z2 - m2) * (z2 - m2), axis=0, keepdims=True)
    u2 = jnp.maximum(g2_ref[...] * (z2 - m2) / jnp.sqrt(v2 + 1e-5) + be2_ref[...], 0.0)
    usc_ref[...] = jnp.dot(u2, w5t_ref[...], preferred_element_type=f32) + b5_ref[...]


@functools.partial(jax.jit, static_argnames=("interpret",))
def kernel(lufeat, llabel, mbank, start, W1, b1, W2, b2, W3, b3, W4, b4,
           W5, b5, g1, be1, g2, be2, interpret=False):
    f32 = jnp.float32
    lbl = llabel.astype(jnp.int32)
    args = (
        lufeat, W1.T, b1.reshape(1, 512), W2.T, b2.reshape(1, 256),
        lbl.reshape(512, 1), lbl.reshape(1, 512), start.reshape(512, 1), mbank,
        W3[:, :256].T, W3[:, 256:].T, b3.reshape(1, 256),
        g1.reshape(1, 256), be1.reshape(1, 256),
        W4.T, b4.reshape(1, 64), g2.reshape(1, 64), be2.reshape(1, 64),
        W5.T, b5.reshape(1, 1),
    )
    lsc, usc, mbu = pl.pallas_call(
        _tc_kernel,
        out_shape=(
            jax.ShapeDtypeStruct((512, 1), f32),
            jax.ShapeDtypeStruct((512, 1), f32),
            jax.ShapeDtypeStruct((512, 256), f32),
        ),
        interpret=interpret,
    )(*args)
    return (lsc.reshape(512), usc, mbu)
